# Initial kernel scaffold; baseline (speedup 1.0000x reference)
import jax
import jax.numpy as jnp
from jax.experimental import pallas as pl


def kernel(x, pos, edge_index, batch, params):
    raise NotImplementedError("write your pallas kernel here")


# speedup vs baseline: 1.7527x; 1.7527x over previous
import functools

import jax
import jax.numpy as jnp
from jax.experimental import pallas as pl
from jax.experimental.pallas import tpu as pltpu

N0 = 10000
E = 320000
DFEAT = 128
HIDDEN = 64
NCLS = 40
EPS = 1e-5
NEG = -1e30

EBLK = 4000


def _edge_mlp_body(g_ref, d_ref, w1d_ref, w2_ref, w3_ref, c_ref, inv_ref, o_ref):
    inv = inv_ref[0]
    d = d_ref[...] * inv
    valid = d_ref[:, 3:4] > 0.5
    h = g_ref[...] + jnp.dot(d, w1d_ref[...], preferred_element_type=jnp.float32)
    c = c_ref[...]
    h = jnp.maximum(h, 0.0) * c[0:1, :] + c[1:2, :]
    h = jnp.dot(h, w2_ref[...], preferred_element_type=jnp.float32) + c[2:3, :]
    h = jnp.maximum(h, 0.0) * c[3:4, :] + c[4:5, :]
    h = jnp.dot(h, w3_ref[...], preferred_element_type=jnp.float32) + c[5:6, :]
    h = jnp.maximum(h, 0.0) * c[6:7, :] + c[7:8, :]
    o_ref[...] = jnp.where(valid, h, NEG)


def _edge_mlp(g, d4, w1d, w2, w3, consts, inv_nrm):
    ne = g.shape[0]
    grid = ne // EBLK
    return pl.pallas_call(
        _edge_mlp_body,
        grid=(grid,),
        in_specs=[
            pl.BlockSpec((EBLK, HIDDEN), lambda i: (i, 0)),
            pl.BlockSpec((EBLK, 8), lambda i: (i, 0)),
            pl.BlockSpec((8, HIDDEN), lambda i: (0, 0)),
            pl.BlockSpec((HIDDEN, HIDDEN), lambda i: (0, 0)),
            pl.BlockSpec((HIDDEN, HIDDEN), lambda i: (0, 0)),
            pl.BlockSpec((8, HIDDEN), lambda i: (0, 0)),
            pl.BlockSpec(memory_space=pltpu.SMEM),
        ],
        out_specs=pl.BlockSpec((EBLK, HIDDEN), lambda i: (i, 0)),
        out_shape=jax.ShapeDtypeStruct((ne, HIDDEN), jnp.float32),
    )(g, d4, w1d, w2, w3, consts, inv_nrm)


def _fold_layers(layers):
    (W1, b1, g1, be1), (W2, b2, g2, be2), (W3, b3, g3, be3) = layers
    s = 1.0 / jnp.sqrt(1.0 + EPS)
    consts = jnp.stack([
        g1 * s, be1,
        b2, g2 * s, be2,
        b3, g3 * s, be3,
    ], axis=0)
    w1d = jnp.zeros((8, HIDDEN), jnp.float32).at[0:3].set(W1[0:3])
    return W1, b1, w1d, W2, W3, consts


def _conv(h, pos, src, dst, valid, layers, kept):
    W1, b1, w1d, W2, W3, consts = _fold_layers(layers)
    y = h @ W1[3:] + b1
    dirv = pos[src] - pos[dst]
    nrm2 = jnp.sum(jnp.where(valid[:, None], dirv * dirv, 0.0))
    inv_nrm = jax.lax.rsqrt(nrm2)
    g = y[dst]
    d4 = jnp.concatenate(
        [dirv, valid[:, None].astype(jnp.float32), jnp.zeros((src.shape[0], 4), jnp.float32)],
        axis=1)
    msg = _edge_mlp(g, d4, w1d, W2, W3, consts,
                    jnp.reshape(inv_nrm, (1,)).astype(jnp.float32))
    out = jax.ops.segment_max(msg, dst, num_segments=N0)
    out = jnp.where(out <= NEG * 0.5, 0.0, out)
    return jnp.where(kept[:, None], out, 0.0)


def _topk_mask(h, kept, k):
    score = jnp.where(kept, jnp.sum(h * h, axis=1), -1.0)
    _, perm = jax.lax.top_k(score, k)
    return jnp.zeros((N0,), bool).at[perm].set(True)


def kernel(x, pos, edge_index, batch, params):
    src, dst = edge_index[0], edge_index[1]
    kept = jnp.ones((N0,), bool)
    valid = jnp.ones((E,), bool)

    h = _conv(x, pos, src, dst, valid, params["conv1"], kept)
    kept = _topk_mask(h, kept, 1800)
    valid = valid & kept[src] & kept[dst]

    h = _conv(h, pos, src, dst, valid, params["conv2"], kept)
    kept = _topk_mask(h, kept, 1400)
    valid = valid & kept[src] & kept[dst]

    h = _conv(h, pos, src, dst, valid, params["conv3"], kept)
    kept = _topk_mask(h, kept, 800)

    g = jnp.sum(jnp.where(kept[:, None], h, 0.0), axis=0, keepdims=True) / 800.0
    W1, b1 = params["lin1"]
    W2, b2 = params["lin2"]
    g = jax.nn.relu(g @ W1 + b1)
    g = g @ W2 + b2
    return jax.nn.log_softmax(g, axis=-1)


# speedup vs baseline: 2.9510x; 1.6837x over previous
import functools

import jax
import jax.numpy as jnp
from jax import lax
from jax.experimental import pallas as pl
from jax.experimental.pallas import tpu as pltpu
from jax.experimental.pallas import tpu_sc as plsc

N0 = 10000
E = 320000
DFEAT = 128
H = 64
NCLS = 40
EPS = 1e-5
NEG = -1e30

NC = 2
NS = 16
NW = NC * NS
L = 16

EBLK = 2000
NCHUNK = E // 128
EPT = E // 4
DBLK = 400
NPAD = 10240


def _node_body(h_refs, pos_ref, kept_ref, w_ref, b_ref,
               y_ref, px_ref, py_ref, pz_ref, kf_ref, *, from_slabs):
    if from_slabs:
        act = h_refs[...]
        act = jnp.where(act <= NEG * 0.5, 0.0, act)
        y_ref[...] = lax.dot_general(
            act, w_ref[...], (((0,), (0,)), ((), ())),
            preferred_element_type=jnp.float32) + b_ref[...]
    else:
        y_ref[...] = (jnp.dot(h_refs[...], w_ref[...],
                              preferred_element_type=jnp.float32) + b_ref[...])
    px_ref[...] = pos_ref[:, 0:1]
    py_ref[...] = pos_ref[:, 1:2]
    pz_ref[...] = pos_ref[:, 2:3]
    kf_ref[...] = kept_ref[...]


def _node_transform(h, pos, kept, w_tail, b1, from_slabs):
    body = functools.partial(_node_body, from_slabs=from_slabs)
    col = jax.ShapeDtypeStruct((N0, 1), jnp.float32)
    row = jax.ShapeDtypeStruct((1, N0), jnp.float32)
    return pl.pallas_call(
        body,
        out_shape=(jax.ShapeDtypeStruct((N0, H), jnp.float32), col, col, col, row),
    )(h, pos, kept, w_tail, b1)


def _gather_body(src_hbm, dst_hbm, px_hbm, py_hbm, pz_hbm, kf_hbm, y_hbm,
                 g_hbm, d8_hbm, nrm_hbm,
                 sidx, didx, bsx, bsy, bsz, bsk, bdx, bdy, bdz, bdk,
                 ry, d8b, accr, sem):
    c = lax.axis_index("c")
    s = lax.axis_index("s")
    wid = s * NC + c

    iot = lax.broadcasted_iota(jnp.int32, (L,), 0)
    zero16 = jnp.zeros((L,), jnp.float32)
    accr[...] = zero16
    for i in range(64):
        d8b[pl.ds(i * 16, 16)] = zero16

    def chunk(j, carry):
        cidx = j * NW + wid

        @pl.when(cidx < NCHUNK)
        def _():
            base = cidx * 128
            pltpu.sync_copy(src_hbm.at[pl.ds(base, 128)], sidx)
            pltpu.sync_copy(dst_hbm.at[pl.ds(base, 128)], didx)
            cps = ((px_hbm, sidx, bsx), (py_hbm, sidx, bsy),
                   (pz_hbm, sidx, bsz), (kf_hbm, sidx, bsk),
                   (px_hbm, didx, bdx), (py_hbm, didx, bdy),
                   (pz_hbm, didx, bdz), (kf_hbm, didx, bdk))
            descs = [pltpu.async_copy(t.at[ix], b, sem) for (t, ix, b) in cps]
            descs.append(pltpu.async_copy(y_hbm.at[didx], ry, sem))
            for dsc in descs:
                dsc.wait()
            acc = accr[...]
            for gi in range(8):
                ds16 = pl.ds(gi * 16, 16)
                ex = bsx[ds16] - bdx[ds16]
                ey = bsy[ds16] - bdy[ds16]
                ez = bsz[ds16] - bdz[ds16]
                val = bsk[ds16] * bdk[ds16]
                acc = acc + val * (ex * ex + ey * ey + ez * ez)
                fl = (iot + gi * 16) * 8
                plsc.store_scatter(d8b, [fl], ex)
                plsc.store_scatter(d8b, [fl + 1], ey)
                plsc.store_scatter(d8b, [fl + 2], ez)
                plsc.store_scatter(d8b, [fl + 3], val)
            accr[...] = acc
            pltpu.sync_copy(ry, g_hbm.at[pl.ds(base, 128)])
            pltpu.sync_copy(d8b, d8_hbm.at[pl.ds(base * 8, 1024)])
        return carry

    lax.fori_loop(0, (NCHUNK + NW - 1) // NW, chunk, 0)
    pltpu.sync_copy(accr, nrm_hbm.at[wid])


@functools.lru_cache(maxsize=None)
def _gather_edges():
    f32buf = pltpu.VMEM((128,), jnp.float32)
    return pl.kernel(
        _gather_body,
        out_type=(jax.ShapeDtypeStruct((E, H), jnp.float32),
                  jax.ShapeDtypeStruct((E * 8,), jnp.float32),
                  jax.ShapeDtypeStruct((NW, L), jnp.float32)),
        mesh=plsc.VectorSubcoreMesh(core_axis_name="c", subcore_axis_name="s"),
        compiler_params=pltpu.CompilerParams(needs_layout_passes=False, use_tc_tiling_on_sc=False),
        scratch_types=[
            pltpu.VMEM((128,), jnp.int32),
            pltpu.VMEM((128,), jnp.int32),
            f32buf, f32buf, f32buf, f32buf, f32buf, f32buf, f32buf, f32buf,
            pltpu.VMEM((128, H), jnp.float32),
            pltpu.VMEM((1024,), jnp.float32),
            pltpu.VMEM((L,), jnp.float32),
            pltpu.SemaphoreType.DMA,
        ],
    )


def _edge_mlp_body(g_ref, d_ref, w1d_ref, w2_ref, w3_ref, c_ref, nrm_ref, o_ref):
    inv = lax.rsqrt(jnp.sum(nrm_ref[...]))
    d = d_ref[...] * inv
    valid = d_ref[:, 3:4] > 0.5
    h = g_ref[...] + jnp.dot(d, w1d_ref[...], preferred_element_type=jnp.float32)
    cc = c_ref[...]
    h = jnp.maximum(h, 0.0) * cc[0:1, :] + cc[1:2, :]
    h = jnp.dot(h, w2_ref[...], preferred_element_type=jnp.float32) + cc[2:3, :]
    h = jnp.maximum(h, 0.0) * cc[3:4, :] + cc[4:5, :]
    h = jnp.dot(h, w3_ref[...], preferred_element_type=jnp.float32) + cc[5:6, :]
    h = jnp.maximum(h, 0.0) * cc[6:7, :] + cc[7:8, :]
    msg = jnp.where(valid, h, NEG)
    for f in range(8):
        o_ref[f] = msg[:, f * 8:(f + 1) * 8]


def _edge_mlp(g, d8, w1d, w2, w3, consts, nrm_parts):
    return pl.pallas_call(
        _edge_mlp_body,
        grid=(E // EBLK,),
        in_specs=[
            pl.BlockSpec((EBLK, H), lambda i: (i, 0)),
            pl.BlockSpec((EBLK, 8), lambda i: (i, 0)),
            pl.BlockSpec((8, H), lambda i: (0, 0)),
            pl.BlockSpec((H, H), lambda i: (0, 0)),
            pl.BlockSpec((H, H), lambda i: (0, 0)),
            pl.BlockSpec((8, H), lambda i: (0, 0)),
            pl.BlockSpec((NW, L), lambda i: (0, 0)),
        ],
        out_specs=pl.BlockSpec((8, EBLK, 8), lambda i: (0, i, 0)),
        out_shape=jax.ShapeDtypeStruct((8, E, 8), jnp.float32),
    )(g, d8, w1d, w2, w3, consts, nrm_parts)


def _scatter_body(msg_hbm, dst_hbm, out_hbm, table, mbuf, dbuf, piece, sbuf, spm, sem):
    c = lax.axis_index("c")
    s = lax.axis_index("s")
    f = c * 4 + s // 4
    k = s % 4
    lf = s // 4

    iot = lax.broadcasted_iota(jnp.int32, (L,), 0)
    lane_feat = jnp.bitwise_and(iot, 7)
    sel = (iot >= 8).astype(jnp.int32)
    perm = jnp.bitwise_and(iot + 8, 15)
    negv = jnp.full((L,), NEG, jnp.float32)

    def initt(i, carry):
        table[pl.ds(i * 16, 16)] = negv
        return carry
    lax.fori_loop(0, (N0 * 8) // 16, initt, 0)

    def blk(i, carry):
        ebase = k * EPT + i * DBLK
        pltpu.sync_copy(msg_hbm.at[f, pl.ds(ebase * 8, DBLK * 8)], mbuf)
        pltpu.sync_copy(dst_hbm.at[pl.ds(ebase, DBLK)], dbuf)

        def pair(p, carry2):
            j2 = p * 2
            ids = plsc.load_gather(dbuf, [j2 + sel])
            oth = plsc.load_gather(dbuf, [j2 + (1 - sel)])
            idx = ids + lane_feat * N0
            mv = mbuf[pl.ds(p * 16, 16)]
            sbuf[...] = mv
            rolled = plsc.load_gather(sbuf, [perm])
            m2 = jnp.maximum(mv, rolled)
            eqf = (ids == oth).astype(jnp.float32)
            mval = mv + eqf * (m2 - mv)
            cur = plsc.load_gather(table, [idx])
            plsc.store_scatter(table, [idx], jnp.maximum(cur, mval))
            return carry2

        lax.fori_loop(0, DBLK // 2, pair, 0)
        return carry

    lax.fori_loop(0, EPT // DBLK, blk, 0)

    for pc in range(10):
        plsc.subcore_barrier()

        @pl.when(k > 0)
        def _():
            pltpu.sync_copy(table.at[pl.ds(pc * 8000, 8000)],
                            spm.at[lf * 3 + (k - 1)])

        plsc.subcore_barrier()

        @pl.when(k == 0)
        def _():
            for other in range(3):
                pltpu.sync_copy(spm.at[lf * 3 + other], piece)

                def mrg(i, carry):
                    off = pc * 8000 + i * 16
                    table[pl.ds(off, 16)] = jnp.maximum(
                        table[pl.ds(off, 16)], piece[pl.ds(i * 16, 16)])
                    return carry
                lax.fori_loop(0, 500, mrg, 0)

    @pl.when(k == 0)
    def _():
        def flo(i, carry):
            v = table[pl.ds(i * 16, 16)]
            table[pl.ds(i * 16, 16)] = jnp.where(v <= NEG * 0.5, 0.0, v)
            return carry
        lax.fori_loop(0, (N0 * 8) // 16, flo, 0)
        pltpu.sync_copy(table, out_hbm.at[f])


@functools.lru_cache(maxsize=None)
def _scatter_max():
    return pl.kernel(
        _scatter_body,
        out_type=jax.ShapeDtypeStruct((8, N0 * 8), jnp.float32),
        mesh=plsc.VectorSubcoreMesh(core_axis_name="c", subcore_axis_name="s"),
        compiler_params=pltpu.CompilerParams(needs_layout_passes=False, use_tc_tiling_on_sc=False),
        scratch_types=[
            pltpu.VMEM((N0 * 8,), jnp.float32),
            pltpu.VMEM((DBLK * 8,), jnp.float32),
            pltpu.VMEM((DBLK,), jnp.int32),
            pltpu.VMEM((8000,), jnp.float32),
            pltpu.VMEM((L,), jnp.float32),
            pltpu.VMEM_SHARED((12, 8000), jnp.float32),
            pltpu.SemaphoreType.DMA,
        ],
    )


def _score_body(s_ref, kept_ref, o_ref):
    act = s_ref[...]
    act = jnp.where(act <= NEG * 0.5, 0.0, act)
    acc = jnp.sum(act * act, axis=0, keepdims=True)
    sc = jnp.where(kept_ref[...] > 0.5, acc, -1.0)
    o_ref[:, 0:N0] = lax.bitcast_convert_type(sc, jnp.int32)
    o_ref[:, N0:NPAD] = jnp.full((1, NPAD - N0), -1, jnp.int32)


def _score(ht, kept):
    return pl.pallas_call(
        _score_body,
        out_shape=jax.ShapeDtypeStruct((1, NPAD), jnp.int32),
    )(ht, kept)


def _bisect_body(s_ref, o_ref, *, kk):
    s = s_ref[...]
    idx = (lax.broadcasted_iota(jnp.int32, (80, 128), 0) * 128
           + lax.broadcasted_iota(jnp.int32, (80, 128), 1))

    def bit_step(i, t):
        cand = jnp.bitwise_or(t, jnp.left_shift(jnp.int32(1), 30 - i))
        cnt = jnp.sum((s >= cand).astype(jnp.int32))
        return jnp.where(cnt >= kk, cand, t)
    tv = lax.fori_loop(0, 31, bit_step, jnp.int32(0))

    n_gt = jnp.sum((s > tv).astype(jnp.int32))
    need = kk - n_gt
    eqm = s == tv

    def idx_step(i, jv):
        cand = jnp.bitwise_or(jv, jnp.left_shift(jnp.int32(1), 13 - i))
        g = jnp.sum((eqm & (idx < cand)).astype(jnp.int32))
        return jnp.where(g < need, cand, jv)
    jmax = lax.fori_loop(0, 14, idx_step, jnp.int32(0))

    ties = eqm & (idx <= jmax) & (need > 0)
    o_ref[...] = ((s > tv) | ties).astype(jnp.float32)


def _topk_mask(ht, kept, kk):
    sbits = jnp.reshape(_score(ht, kept), (80, 128))
    body = functools.partial(_bisect_body, kk=kk)
    keptp = pl.pallas_call(
        body,
        out_shape=jax.ShapeDtypeStruct((80, 128), jnp.float32),
    )(sbits)
    return jnp.reshape(keptp, (1, NPAD))[:, 0:N0]


def _head_body(s_ref, kept_ref, w1_ref, b1_ref, w2_ref, b2_ref, o_ref):
    act = s_ref[...]
    act = jnp.where(act <= NEG * 0.5, 0.0, act)
    act = jnp.where(kept_ref[...] > 0.5, act, 0.0)
    gv = jnp.sum(act, axis=1, keepdims=True)
    g = jnp.transpose(gv) / 800.0
    g = jnp.maximum(jnp.dot(g, w1_ref[...], preferred_element_type=jnp.float32)
                    + b1_ref[...], 0.0)
    z = jnp.dot(g, w2_ref[...], preferred_element_type=jnp.float32) + b2_ref[...]
    m = jnp.max(z, axis=1, keepdims=True)
    zs = z - m
    o_ref[...] = zs - jnp.log(jnp.sum(jnp.exp(zs), axis=1, keepdims=True))


def _head(ht, kept, w1, b1, w2, b2):
    return pl.pallas_call(
        _head_body,
        out_shape=jax.ShapeDtypeStruct((1, NCLS), jnp.float32),
    )(ht, kept, w1, b1, w2, b2)


def _fold_layers(layers):
    (W1, b1, g1, be1), (W2, b2, g2, be2), (W3, b3, g3, be3) = layers
    sc = 1.0 / jnp.sqrt(1.0 + EPS)
    consts = jnp.stack([g1 * sc, be1, b2, g2 * sc, be2, b3, g3 * sc, be3], axis=0)
    w1d = jnp.zeros((8, H), jnp.float32).at[0:3].set(W1[0:3])
    return W1[3:], b1, w1d, W2, W3, consts


def _conv(h, pos, kept, src, dst, layers, from_slabs):
    w_tail, b1, w1d, W2, W3, consts = _fold_layers(layers)
    y, px, py, pz, kf = _node_transform(
        h, pos, kept, w_tail, jnp.reshape(b1, (1, H)), from_slabs)
    g, d8, nrmp = _gather_edges()(
        src, dst, jnp.reshape(px, (N0,)), jnp.reshape(py, (N0,)),
        jnp.reshape(pz, (N0,)), jnp.reshape(kf, (N0,)), y)
    msg = _edge_mlp(g, jnp.reshape(d8, (E, 8)), w1d, W2, W3, consts, nrmp)
    slabs = _scatter_max()(jnp.reshape(msg, (8, E * 8)), dst)
    return jnp.reshape(slabs, (H, N0))


def kernel(x, pos, edge_index, batch, params):
    src = edge_index[0].astype(jnp.int32)
    dst = edge_index[1].astype(jnp.int32)
    kept = jnp.ones((1, N0), jnp.float32)

    ht = _conv(x, pos, kept, src, dst, params["conv1"], from_slabs=False)
    kept = _topk_mask(ht, kept, 1800)
    ht = _conv(ht, pos, kept, src, dst, params["conv2"], from_slabs=True)
    kept = _topk_mask(ht, kept, 1400)
    ht = _conv(ht, pos, kept, src, dst, params["conv3"], from_slabs=True)
    kept = _topk_mask(ht, kept, 800)

    W1l, b1l = params["lin1"]
    W2l, b2l = params["lin2"]
    return _head(ht, kept, W1l, jnp.reshape(b1l, (1, H)),
                 W2l, jnp.reshape(b2l, (1, NCLS)))
